# baseline (speedup 1.0000x reference)
import jax
import jax.numpy as jnp
from jax import lax
from jax.experimental import pallas as pl
from jax.experimental.pallas import tpu as pltpu, tpu_sc as plsc

_N_FIELDS = 26
_VOCAB = 100000
_V_PAD = 102400
_EMB_DIM = 16
_BATCH = 16384

_NC = 2
_NS = 16
_NW = _NC * _NS

_B_PER_W = _BATCH // _NW
_IDX_MINOR = 128
_SUBS = _B_PER_W // _IDX_MINOR
_GATHER_BATCH = 13

_V_CHUNK = _V_PAD
_F_CHUNK = 2


def _tc_reduce_body(t_ref, o_ref):
    for i in range(_F_CHUNK):
        o_ref[pl.ds(i * _V_PAD, _V_PAD)] = jnp.sum(t_ref[i], axis=0)


def _sc_body(s_hbm, x_hbm, w_hbm, out_hbm, idx_v, g_v, w_v, out_v, sem):
    wid = lax.axis_index("s") * _NC + lax.axis_index("c")

    pltpu.sync_copy(x_hbm.at[:, pl.ds(wid * _SUBS, _SUBS), :], idx_v)
    pltpu.sync_copy(w_hbm, w_v)
    w_regs = [w_v[f] for f in range(_N_FIELDS)]

    def fire(f):
        return [pltpu.async_copy(
            s_hbm.at[f].at[idx_v.at[f, s]],
            g_v.at[f, pl.ds(s * _IDX_MINOR, _IDX_MINOR)],
            sem) for s in range(_SUBS)]

    ahead = 3
    copies = []
    for f in range(min(ahead, _N_FIELDS)):
        copies.append(fire(f))

    for f in range(_N_FIELDS):
        for cp in copies[f]:
            cp.wait()
        nxt = f + ahead
        if nxt < _N_FIELDS:
            copies.append(fire(nxt))
        w_f = w_regs[f]

        def group(g16, carry):
            b0 = g16 * 16
            acc = g_v[f, pl.ds(b0, 16)] * w_f
            if f > 0:
                acc = acc + out_v[pl.ds(b0, 16)]
            out_v[pl.ds(b0, 16)] = acc
            return carry

        lax.fori_loop(0, _B_PER_W // 16, group, 0)

    pltpu.sync_copy(out_v, out_hbm.at[pl.ds(wid * _B_PER_W, _B_PER_W)])


@jax.jit
def _run(tbl_t, xt, wmat):
    s_flat = pl.pallas_call(
        _tc_reduce_body,
        grid=(_N_FIELDS // _F_CHUNK,),
        in_specs=[pl.BlockSpec((_F_CHUNK, _EMB_DIM, _V_CHUNK),
                               lambda f: (f, 0, 0))],
        out_specs=pl.BlockSpec((_F_CHUNK * _V_PAD,), lambda f: (f,)),
        out_shape=jax.ShapeDtypeStruct((_N_FIELDS * _V_PAD,), jnp.float32),
    )(tbl_t)
    s2 = s_flat.reshape(_N_FIELDS, _V_PAD)

    mesh = plsc.VectorSubcoreMesh(core_axis_name="c", subcore_axis_name="s")
    f = pl.kernel(
        _sc_body,
        mesh=mesh,
        compiler_params=pltpu.CompilerParams(
            needs_layout_passes=False, use_tc_tiling_on_sc=False),
        out_type=jax.ShapeDtypeStruct((_BATCH,), jnp.float32),
        scratch_types=[
            pltpu.VMEM((_N_FIELDS, _SUBS, _IDX_MINOR), jnp.int32),
            pltpu.VMEM((_N_FIELDS, _B_PER_W), jnp.float32),
            pltpu.VMEM((_N_FIELDS, _EMB_DIM), jnp.float32),
            pltpu.VMEM((_B_PER_W,), jnp.float32),
            pltpu.SemaphoreType.DMA,
        ],
    )
    return f(s2, xt, wmat)


def kernel(X, tables, alpha):
    w = jnp.tanh(alpha).astype(jnp.float32)
    wmat = jnp.broadcast_to(w[:, None], (_N_FIELDS, _EMB_DIM))
    tbl_t = jnp.transpose(tables, (0, 2, 1))
    xt = jnp.transpose(X, (1, 0)).reshape(-1).reshape(
        _N_FIELDS, _BATCH // _IDX_MINOR, _IDX_MINOR)
    out = _run(tbl_t, xt, wmat)
    return out[:, None]


# speedup vs baseline: 1.0258x; 1.0258x over previous
import jax
import jax.numpy as jnp
from jax import lax
from jax.experimental import pallas as pl
from jax.experimental.pallas import tpu as pltpu, tpu_sc as plsc

_N_FIELDS = 26
_F_SPLIT = 18
_VOCAB = 100000
_V_PAD = 102400
_EMB_DIM = 16
_BATCH = 16384

_NC = 2
_NS = 16
_NW = _NC * _NS

_B_PER_W = _BATCH // _NW
_IDX_MINOR = 128
_SUBS = _B_PER_W // _IDX_MINOR

_F_CHUNK = 2


def _tc_reduce_body(t_ref, o_ref):
    for i in range(_F_CHUNK):
        o_ref[pl.ds(i * _V_PAD, _V_PAD)] = jnp.sum(t_ref[i], axis=0)


def _tc_reduce(tbl_t, f_lo, f_cnt):
    return pl.pallas_call(
        _tc_reduce_body,
        grid=(f_cnt // _F_CHUNK,),
        in_specs=[pl.BlockSpec((_F_CHUNK, _EMB_DIM, _V_PAD),
                               lambda f: (f_lo // _F_CHUNK + f, 0, 0))],
        out_specs=pl.BlockSpec((_F_CHUNK * _V_PAD,), lambda f: (f,)),
        out_shape=jax.ShapeDtypeStruct((f_cnt * _V_PAD,), jnp.float32),
    )(tbl_t).reshape(f_cnt, _V_PAD)


def _make_sc_body(f_lo, f_cnt, has_partial):
    def body(*refs):
        if has_partial:
            (s_hbm, x_hbm, w_hbm, p_hbm, out_hbm,
             idx_v, g_v, w_v, out_v, sem) = refs
        else:
            (s_hbm, x_hbm, w_hbm, out_hbm,
             idx_v, g_v, w_v, out_v, sem) = refs
        wid = lax.axis_index("s") * _NC + lax.axis_index("c")

        pltpu.sync_copy(
            x_hbm.at[pl.ds(f_lo, f_cnt), pl.ds(wid * _SUBS, _SUBS), :], idx_v)
        pltpu.sync_copy(w_hbm, w_v)
        if has_partial:
            pltpu.sync_copy(p_hbm.at[pl.ds(wid * _B_PER_W, _B_PER_W)], out_v)
        w_regs = [w_v[f_lo + f] for f in range(f_cnt)]

        def fire(f):
            return [pltpu.async_copy(
                s_hbm.at[f].at[idx_v.at[f, s]],
                g_v.at[f, pl.ds(s * _IDX_MINOR, _IDX_MINOR)],
                sem) for s in range(_SUBS)]

        ahead = 3
        copies = []
        for f in range(min(ahead, f_cnt)):
            copies.append(fire(f))

        for f in range(f_cnt):
            for cp in copies[f]:
                cp.wait()
            if f + ahead < f_cnt:
                copies.append(fire(f + ahead))
            w_f = w_regs[f]
            first = (f == 0) and not has_partial

            def group(g16, carry):
                b0 = g16 * 16
                acc = g_v[f, pl.ds(b0, 16)] * w_f
                if not first:
                    acc = acc + out_v[pl.ds(b0, 16)]
                out_v[pl.ds(b0, 16)] = acc
                return carry

            lax.fori_loop(0, _B_PER_W // 16, group, 0)

        pltpu.sync_copy(out_v, out_hbm.at[pl.ds(wid * _B_PER_W, _B_PER_W)])

    return body


def _sc_gather(s2, xt, wmat, f_lo, f_cnt, partial=None):
    mesh = plsc.VectorSubcoreMesh(core_axis_name="c", subcore_axis_name="s")
    f = pl.kernel(
        _make_sc_body(f_lo, f_cnt, partial is not None),
        mesh=mesh,
        compiler_params=pltpu.CompilerParams(
            needs_layout_passes=False, use_tc_tiling_on_sc=False),
        out_type=jax.ShapeDtypeStruct((_BATCH,), jnp.float32),
        scratch_types=[
            pltpu.VMEM((f_cnt, _SUBS, _IDX_MINOR), jnp.int32),
            pltpu.VMEM((f_cnt, _B_PER_W), jnp.float32),
            pltpu.VMEM((_N_FIELDS, _EMB_DIM), jnp.float32),
            pltpu.VMEM((_B_PER_W,), jnp.float32),
            pltpu.SemaphoreType.DMA,
        ],
    )
    args = (s2, xt, wmat) + (() if partial is None else (partial,))
    return f(*args)


@jax.jit
def _run(tbl_t, xt, wmat):
    s_a = _tc_reduce(tbl_t, 0, _F_SPLIT)
    s_b = _tc_reduce(tbl_t, _F_SPLIT, _N_FIELDS - _F_SPLIT)
    part = _sc_gather(s_a, xt, wmat, 0, _F_SPLIT)
    out = _sc_gather(s_b, xt, wmat, _F_SPLIT, _N_FIELDS - _F_SPLIT,
                     partial=part)
    return out


def kernel(X, tables, alpha):
    w = jnp.tanh(alpha).astype(jnp.float32)
    wmat = jnp.broadcast_to(w[:, None], (_N_FIELDS, _EMB_DIM))
    tbl_t = jnp.transpose(tables, (0, 2, 1))
    xt = jnp.transpose(X, (1, 0)).reshape(-1).reshape(
        _N_FIELDS, _BATCH // _IDX_MINOR, _IDX_MINOR)
    out = _run(tbl_t, xt, wmat)
    return out[:, None]


# speedup vs baseline: 1.0580x; 1.0315x over previous
import jax
import jax.numpy as jnp
from jax import lax
from jax.experimental import pallas as pl
from jax.experimental.pallas import tpu as pltpu, tpu_sc as plsc

_N_FIELDS = 26
_F_SPLIT = 16
_VOCAB = 100000
_V_PAD = 102400
_EMB_DIM = 16
_BATCH = 16384

_NC = 2
_NS = 16
_NW = _NC * _NS

_B_PER_W = _BATCH // _NW
_IDX_MINOR = 128
_SUBS = _B_PER_W // _IDX_MINOR

_F_CHUNK = 2


def _tc_reduce_body(t_ref, o_ref):
    for i in range(_F_CHUNK):
        o_ref[pl.ds(i * _V_PAD, _V_PAD)] = jnp.sum(t_ref[i], axis=0)


def _tc_reduce(tbl_t, f_lo, f_cnt):
    return pl.pallas_call(
        _tc_reduce_body,
        grid=(f_cnt // _F_CHUNK,),
        in_specs=[pl.BlockSpec((_F_CHUNK, _EMB_DIM, _V_PAD),
                               lambda f: (f_lo // _F_CHUNK + f, 0, 0))],
        out_specs=pl.BlockSpec((_F_CHUNK * _V_PAD,), lambda f: (f,)),
        out_shape=jax.ShapeDtypeStruct((f_cnt * _V_PAD,), jnp.float32),
    )(tbl_t).reshape(f_cnt, _V_PAD)


def _make_sc_body(f_lo, f_cnt, has_partial):
    def body(*refs):
        if has_partial:
            (s_hbm, x_hbm, w_hbm, p_hbm, out_hbm,
             idx_v, g_v, w_v, out_v, sem) = refs
        else:
            (s_hbm, x_hbm, w_hbm, out_hbm,
             idx_v, g_v, w_v, out_v, sem) = refs
        wid = lax.axis_index("s") * _NC + lax.axis_index("c")

        pltpu.sync_copy(
            x_hbm.at[pl.ds(f_lo, f_cnt), pl.ds(wid * _SUBS, _SUBS), :], idx_v)
        pltpu.sync_copy(w_hbm, w_v)
        if has_partial:
            pltpu.sync_copy(p_hbm.at[pl.ds(wid * _B_PER_W, _B_PER_W)], out_v)
        w_regs = [w_v[f_lo + f] for f in range(f_cnt)]

        def fire(f):
            return [pltpu.async_copy(
                s_hbm.at[f].at[idx_v.at[f, s]],
                g_v.at[f, pl.ds(s * _IDX_MINOR, _IDX_MINOR)],
                sem) for s in range(_SUBS)]

        ahead = 5
        copies = []
        for f in range(min(ahead, f_cnt)):
            copies.append(fire(f))

        for f in range(f_cnt):
            for cp in copies[f]:
                cp.wait()
            if f + ahead < f_cnt:
                copies.append(fire(f + ahead))
            w_f = w_regs[f]
            first = (f == 0) and not has_partial

            def group(g16, carry):
                b0 = g16 * 16
                acc = g_v[f, pl.ds(b0, 16)] * w_f
                if not first:
                    acc = acc + out_v[pl.ds(b0, 16)]
                out_v[pl.ds(b0, 16)] = acc
                return carry

            lax.fori_loop(0, _B_PER_W // 16, group, 0)

        pltpu.sync_copy(out_v, out_hbm.at[pl.ds(wid * _B_PER_W, _B_PER_W)])

    return body


def _sc_gather(s2, xt, wmat, f_lo, f_cnt, partial=None):
    mesh = plsc.VectorSubcoreMesh(core_axis_name="c", subcore_axis_name="s")
    f = pl.kernel(
        _make_sc_body(f_lo, f_cnt, partial is not None),
        mesh=mesh,
        compiler_params=pltpu.CompilerParams(
            needs_layout_passes=False, use_tc_tiling_on_sc=False),
        out_type=jax.ShapeDtypeStruct((_BATCH,), jnp.float32),
        scratch_types=[
            pltpu.VMEM((f_cnt, _SUBS, _IDX_MINOR), jnp.int32),
            pltpu.VMEM((f_cnt, _B_PER_W), jnp.float32),
            pltpu.VMEM((_N_FIELDS, _EMB_DIM), jnp.float32),
            pltpu.VMEM((_B_PER_W,), jnp.float32),
            pltpu.SemaphoreType.DMA,
        ],
    )
    args = (s2, xt, wmat) + (() if partial is None else (partial,))
    return f(*args)


@jax.jit
def _run(tbl_t, xt, wmat):
    s_a = _tc_reduce(tbl_t, 0, _F_SPLIT)
    s_b = _tc_reduce(tbl_t, _F_SPLIT, _N_FIELDS - _F_SPLIT)
    part = _sc_gather(s_a, xt, wmat, 0, _F_SPLIT)
    out = _sc_gather(s_b, xt, wmat, _F_SPLIT, _N_FIELDS - _F_SPLIT,
                     partial=part)
    return out


def kernel(X, tables, alpha):
    w = jnp.tanh(alpha).astype(jnp.float32)
    wmat = jnp.broadcast_to(w[:, None], (_N_FIELDS, _EMB_DIM))
    tbl_t = jnp.transpose(tables, (0, 2, 1))
    xt = jnp.transpose(X, (1, 0)).reshape(-1).reshape(
        _N_FIELDS, _BATCH // _IDX_MINOR, _IDX_MINOR)
    out = _run(tbl_t, xt, wmat)
    return out[:, None]
